# baseline (speedup 1.0000x reference)
import jax
import jax.numpy as jnp
from jax import lax
from jax.experimental import pallas as pl
from jax.experimental.pallas import tpu as pltpu
from jax.experimental.pallas import tpu_sc as plsc

TOP_K = 2
NUM_EXPERTS = 8
DIM = 768
ALPHA = 0.001
BSZ = 4
SEQ = 8192

TOKENS = BSZ * SEQ
NCHUNK = 2
CTOK = TOKENS // NCHUNK
LANES = 16
NCORES = 2
NSUB = 16
NW = NCORES * NSUB
TPW = CTOK // NW
GROUPS = TPW // LANES
NACC = 2 * NUM_EXPERTS
NEG_INF = float("-inf")

BLOCK_T = 2048
GRID = CTOK // BLOCK_T

AUX_SCALE = ALPHA * NUM_EXPERTS / (SEQ * TOP_K) / SEQ / BSZ


def _scores_body(x_ref, w_ref, s_ref):
    x = x_ref[...]
    w = w_ref[...]
    logits = lax.dot_general(
        w, x, (((1,), (1,)), ((), ())),
        preferred_element_type=jnp.float32)
    m = jnp.max(logits, axis=0, keepdims=True)
    ex = jnp.exp(logits - m)
    s_ref[...] = ex / jnp.sum(ex, axis=0, keepdims=True)


def _tc_scores(hs_chunk, weight):
    return pl.pallas_call(
        _scores_body,
        grid=(GRID,),
        in_specs=[
            pl.BlockSpec((BLOCK_T, DIM), lambda i: (i, 0)),
            pl.BlockSpec((NUM_EXPERTS, DIM), lambda i: (0, 0)),
        ],
        out_specs=pl.BlockSpec((NUM_EXPERTS, BLOCK_T), lambda i: (0, i)),
        out_shape=jax.ShapeDtypeStruct((NUM_EXPERTS, CTOK), jnp.float32),
    )(hs_chunk, weight)


def _routing_body(scores_hbm, it_hbm, wt_hbm, aux_hbm,
                  scores_v, i1_v, i2_v, w1_v, w2_v, acc_v, red_v, out16_v,
                  shared, dma_sem):
    cid = lax.axis_index("c")
    sid = lax.axis_index("s")
    wid = cid * NSUB + sid
    base = wid * TPW

    copies = [
        pltpu.async_copy(scores_hbm.at[e, pl.ds(base, TPW)],
                         scores_v.at[pl.ds(e * TPW, TPW)], dma_sem)
        for e in range(NUM_EXPERTS)
    ]
    for c in copies:
        c.wait()

    lane = lax.iota(jnp.int32, LANES)
    zf = jnp.zeros((LANES,), jnp.float32)

    def group(g, acc):
        cnt, ssum = acc
        off = g * LANES
        p = [scores_v[pl.ds(e * TPW + off, LANES)] for e in range(NUM_EXPERTS)]

        m1 = p[0]
        i1 = jnp.zeros((LANES,), jnp.int32)
        m2 = jnp.full((LANES,), NEG_INF, jnp.float32)
        i2 = jnp.zeros((LANES,), jnp.int32)
        for e in range(1, NUM_EXPERTS):
            pe = p[e]
            ei = jnp.full((LANES,), e, jnp.int32)
            gt1 = pe > m1
            gt2 = pe > m2
            i2 = jnp.where(gt1, i1, jnp.where(gt2, ei, i2))
            m2 = jnp.where(gt1, m1, jnp.where(gt2, pe, m2))
            i1 = jnp.where(gt1, ei, i1)
            m1 = jnp.where(gt1, pe, m1)

        r = 1.0 / (m1 + m2 + 1e-20)
        sl = pl.ds(off, LANES)
        i1_v[sl] = i1
        i2_v[sl] = i2
        w1_v[sl] = m1 * r
        w2_v[sl] = m2 * r

        cnt = [cnt[e]
               + jnp.where(i1 == e, 1.0, zf)
               + jnp.where(i2 == e, 1.0, zf)
               for e in range(NUM_EXPERTS)]
        ssum = [ssum[e] + p[e] for e in range(NUM_EXPERTS)]
        return (cnt, ssum)

    init = ([zf] * NUM_EXPERTS, [zf] * NUM_EXPERTS)
    cnt, ssum = lax.fori_loop(0, GROUPS, group, init)

    pltpu.sync_copy(i1_v, it_hbm.at[0, pl.ds(base, TPW)])
    pltpu.sync_copy(i2_v, it_hbm.at[1, pl.ds(base, TPW)])
    pltpu.sync_copy(w1_v, wt_hbm.at[0, pl.ds(base, TPW)])
    pltpu.sync_copy(w2_v, wt_hbm.at[1, pl.ds(base, TPW)])

    for e in range(NUM_EXPERTS):
        acc_v[e, :] = cnt[e]
        acc_v[NUM_EXPERTS + e, :] = ssum[e]
    pltpu.sync_copy(acc_v, shared.at[sid])
    plsc.subcore_barrier()

    @pl.when(sid == 0)
    def _reduce():
        pltpu.sync_copy(shared, red_v)
        tot = []
        for a in range(NACC):
            v = red_v[0, a, :]
            for w in range(1, NSUB):
                v = v + red_v[w, a, :]
            tot.append(jnp.sum(v))
        term = tot[0] * tot[NUM_EXPERTS]
        for e in range(1, NUM_EXPERTS):
            term = term + tot[e] * tot[NUM_EXPERTS + e]
        out16_v[...] = jnp.where(lane == 0, term * AUX_SCALE, zf)
        pltpu.sync_copy(out16_v, aux_hbm.at[pl.ds(cid * LANES, LANES)])


_sc_routing = pl.kernel(
    _routing_body,
    out_type=(
        jax.ShapeDtypeStruct((TOP_K, CTOK), jnp.int32),
        jax.ShapeDtypeStruct((TOP_K, CTOK), jnp.float32),
        jax.ShapeDtypeStruct((NCORES * LANES,), jnp.float32),
    ),
    mesh=plsc.VectorSubcoreMesh(core_axis_name="c", subcore_axis_name="s"),
    compiler_params=pltpu.CompilerParams(needs_layout_passes=False),
    scratch_types=[
        pltpu.VMEM((NUM_EXPERTS * TPW,), jnp.float32),
        pltpu.VMEM((TPW,), jnp.int32),
        pltpu.VMEM((TPW,), jnp.int32),
        pltpu.VMEM((TPW,), jnp.float32),
        pltpu.VMEM((TPW,), jnp.float32),
        pltpu.VMEM((NACC, LANES), jnp.float32),
        pltpu.VMEM((NSUB, NACC, LANES), jnp.float32),
        pltpu.VMEM((LANES,), jnp.float32),
        pltpu.VMEM_SHARED((NSUB, NACC, LANES), jnp.float32),
        pltpu.SemaphoreType.DMA,
    ],
)


FMT_GRID = 16
FMT_T = TOKENS // FMT_GRID
FMT_STEPS_A = CTOK // FMT_T


def _format_body(ita_ref, itb_ref, wta_ref, wtb_ref, auxa_ref, auxb_ref,
                 io_ref, wo_ref, ao_ref):
    i = pl.program_id(0)
    in_a = i < FMT_STEPS_A
    it = jnp.where(in_a, ita_ref[...], itb_ref[...])
    wt = jnp.where(in_a, wta_ref[...], wtb_ref[...])
    io_ref[...] = jnp.transpose(it)
    wo_ref[...] = jnp.transpose(wt)

    @pl.when(i == 0)
    def _aux():
        ao_ref[...] = (jnp.sum(auxa_ref[...], axis=1, keepdims=True)
                       + jnp.sum(auxb_ref[...], axis=1, keepdims=True))


def _tc_format(ita, itb, wta, wtb, auxa, auxb):
    half = FMT_STEPS_A

    def a_map(i):
        return (0, jnp.minimum(i, half - 1))

    def b_map(i):
        return (0, jnp.maximum(i - half, 0))

    return pl.pallas_call(
        _format_body,
        grid=(FMT_GRID,),
        in_specs=[
            pl.BlockSpec((TOP_K, FMT_T), a_map),
            pl.BlockSpec((TOP_K, FMT_T), b_map),
            pl.BlockSpec((TOP_K, FMT_T), a_map),
            pl.BlockSpec((TOP_K, FMT_T), b_map),
            pl.BlockSpec((1, NCORES * LANES), lambda i: (0, 0)),
            pl.BlockSpec((1, NCORES * LANES), lambda i: (0, 0)),
        ],
        out_specs=(
            pl.BlockSpec((FMT_T, TOP_K), lambda i: (i, 0)),
            pl.BlockSpec((FMT_T, TOP_K), lambda i: (i, 0)),
            pl.BlockSpec((1, 1), lambda i: (0, 0)),
        ),
        out_shape=(
            jax.ShapeDtypeStruct((TOKENS, TOP_K), jnp.int32),
            jax.ShapeDtypeStruct((TOKENS, TOP_K), jnp.float32),
            jax.ShapeDtypeStruct((1, 1), jnp.float32),
        ),
    )(ita, itb, wta, wtb, auxa, auxb)


@jax.jit
def kernel(hidden_states, weight):
    hs = hidden_states.reshape(TOKENS, DIM)
    sc_a = _tc_scores(hs[:CTOK], weight)
    sc_b = _tc_scores(hs[CTOK:], weight)
    ita, wta, auxa = _sc_routing(sc_a)
    itb, wtb, auxb = _sc_routing(sc_b)
    topk_idx, topk_w, aux = _tc_format(
        ita, itb, wta, wtb,
        auxa.reshape(1, NCORES * LANES), auxb.reshape(1, NCORES * LANES))
    return (topk_idx, topk_w, aux[0, 0])


# speedup vs baseline: 1.5698x; 1.5698x over previous
import jax
import jax.numpy as jnp
from jax import lax
from jax.experimental import pallas as pl
from jax.experimental.pallas import tpu as pltpu
from jax.experimental.pallas import tpu_sc as plsc

TOP_K = 2
NUM_EXPERTS = 8
DIM = 768
ALPHA = 0.001
BSZ = 4
SEQ = 8192

TOKENS = BSZ * SEQ
LANES = 16
NCORES = 2
NSUB = 16
NW = NCORES * NSUB
TPW = TOKENS // NW
GROUPS = TPW // LANES
NACC = 2 * NUM_EXPERTS
NEG_INF = float("-inf")

BLOCK_T = 2048
GRID = TOKENS // BLOCK_T

AUX_SCALE = ALPHA * NUM_EXPERTS / (SEQ * TOP_K) / SEQ / BSZ


def _scores_body(x_ref, w_ref, s_ref):
    x = x_ref[...]
    w = w_ref[...]
    logits = lax.dot_general(
        w, x, (((1,), (1,)), ((), ())),
        preferred_element_type=jnp.float32)
    m = jnp.max(logits, axis=0, keepdims=True)
    ex = jnp.exp(logits - m)
    s_ref[...] = ex / jnp.sum(ex, axis=0, keepdims=True)


def _tc_scores(hs, weight):
    return pl.pallas_call(
        _scores_body,
        grid=(GRID,),
        in_specs=[
            pl.BlockSpec((BLOCK_T, DIM), lambda i: (i, 0)),
            pl.BlockSpec((NUM_EXPERTS, DIM), lambda i: (0, 0)),
        ],
        out_specs=pl.BlockSpec((NUM_EXPERTS, BLOCK_T), lambda i: (0, i)),
        out_shape=jax.ShapeDtypeStruct((NUM_EXPERTS, TOKENS), jnp.float32),
    )(hs, weight)


def _routing_body(scores_hbm, it_hbm, wt_hbm, aux_hbm,
                  scores_v, i1_v, i2_v, w1_v, w2_v, acc_v, red_v, out16_v,
                  shared, dma_sem):
    cid = lax.axis_index("c")
    sid = lax.axis_index("s")
    wid = cid * NSUB + sid
    base = wid * TPW

    copies = [
        pltpu.async_copy(scores_hbm.at[e, pl.ds(base, TPW)],
                         scores_v.at[pl.ds(e * TPW, TPW)], dma_sem)
        for e in range(NUM_EXPERTS)
    ]
    for c in copies:
        c.wait()

    lane = lax.iota(jnp.int32, LANES)
    zf = jnp.zeros((LANES,), jnp.float32)

    def group(g, acc):
        cnt, ssum = acc
        off = g * LANES
        p = [scores_v[pl.ds(e * TPW + off, LANES)] for e in range(NUM_EXPERTS)]

        m1 = p[0]
        i1 = jnp.zeros((LANES,), jnp.int32)
        m2 = jnp.full((LANES,), NEG_INF, jnp.float32)
        i2 = jnp.zeros((LANES,), jnp.int32)
        for e in range(1, NUM_EXPERTS):
            pe = p[e]
            ei = jnp.full((LANES,), e, jnp.int32)
            gt1 = pe > m1
            gt2 = pe > m2
            i2 = jnp.where(gt1, i1, jnp.where(gt2, ei, i2))
            m2 = jnp.where(gt1, m1, jnp.where(gt2, pe, m2))
            i1 = jnp.where(gt1, ei, i1)
            m1 = jnp.where(gt1, pe, m1)

        r = 1.0 / (m1 + m2 + 1e-20)
        sl = pl.ds(off, LANES)
        i1_v[sl] = i1
        i2_v[sl] = i2
        w1_v[sl] = m1 * r
        w2_v[sl] = m2 * r

        cnt = [cnt[e]
               + jnp.where(i1 == e, 1.0, zf)
               + jnp.where(i2 == e, 1.0, zf)
               for e in range(NUM_EXPERTS)]
        ssum = [ssum[e] + p[e] for e in range(NUM_EXPERTS)]
        return (cnt, ssum)

    init = ([zf] * NUM_EXPERTS, [zf] * NUM_EXPERTS)
    cnt, ssum = lax.fori_loop(0, GROUPS, group, init)

    pltpu.sync_copy(i1_v, it_hbm.at[0, pl.ds(base, TPW)])
    pltpu.sync_copy(i2_v, it_hbm.at[1, pl.ds(base, TPW)])
    pltpu.sync_copy(w1_v, wt_hbm.at[0, pl.ds(base, TPW)])
    pltpu.sync_copy(w2_v, wt_hbm.at[1, pl.ds(base, TPW)])

    for e in range(NUM_EXPERTS):
        acc_v[e, :] = cnt[e]
        acc_v[NUM_EXPERTS + e, :] = ssum[e]
    pltpu.sync_copy(acc_v, shared.at[sid])
    plsc.subcore_barrier()

    @pl.when(sid == 0)
    def _reduce():
        pltpu.sync_copy(shared, red_v)
        aux = jnp.float32(0.0)
        for b in range(2):
            tot = []
            for a in range(NACC):
                v = red_v[8 * b, a, :]
                for w in range(8 * b + 1, 8 * b + 8):
                    v = v + red_v[w, a, :]
                tot.append(jnp.sum(v))
            term = tot[0] * tot[NUM_EXPERTS]
            for e in range(1, NUM_EXPERTS):
                term = term + tot[e] * tot[NUM_EXPERTS + e]
            aux = aux + term
        out16_v[...] = jnp.where(lane == 0, aux * AUX_SCALE, zf)
        pltpu.sync_copy(out16_v, aux_hbm.at[pl.ds(cid * LANES, LANES)])


_sc_routing = pl.kernel(
    _routing_body,
    out_type=(
        jax.ShapeDtypeStruct((TOP_K, TOKENS), jnp.int32),
        jax.ShapeDtypeStruct((TOP_K, TOKENS), jnp.float32),
        jax.ShapeDtypeStruct((NCORES * LANES,), jnp.float32),
    ),
    mesh=plsc.VectorSubcoreMesh(core_axis_name="c", subcore_axis_name="s"),
    compiler_params=pltpu.CompilerParams(needs_layout_passes=False),
    scratch_types=[
        pltpu.VMEM((NUM_EXPERTS * TPW,), jnp.float32),
        pltpu.VMEM((TPW,), jnp.int32),
        pltpu.VMEM((TPW,), jnp.int32),
        pltpu.VMEM((TPW,), jnp.float32),
        pltpu.VMEM((TPW,), jnp.float32),
        pltpu.VMEM((NACC, LANES), jnp.float32),
        pltpu.VMEM((NSUB, NACC, LANES), jnp.float32),
        pltpu.VMEM((LANES,), jnp.float32),
        pltpu.VMEM_SHARED((NSUB, NACC, LANES), jnp.float32),
        pltpu.SemaphoreType.DMA,
    ],
)


FMT_GRID = 32
FMT_T = TOKENS // FMT_GRID
FMT_BR = FMT_T // 128


def _format_body(it_ref, wt_ref, aux_ref, io_ref, wo_ref, ao_ref):
    io_ref[...] = jnp.transpose(it_ref[...])
    wo_ref[...] = jnp.transpose(wt_ref[...])

    @pl.when(pl.program_id(0) == 0)
    def _aux():
        ao_ref[...] = jnp.sum(aux_ref[...], axis=1, keepdims=True)


def _tc_format(it, wt, aux2):
    return pl.pallas_call(
        _format_body,
        grid=(FMT_GRID,),
        in_specs=[
            pl.BlockSpec((TOP_K, FMT_T), lambda i: (0, i)),
            pl.BlockSpec((TOP_K, FMT_T), lambda i: (0, i)),
            pl.BlockSpec((1, NCORES * LANES), lambda i: (0, 0)),
        ],
        out_specs=(
            pl.BlockSpec((FMT_T, TOP_K), lambda i: (i, 0)),
            pl.BlockSpec((FMT_T, TOP_K), lambda i: (i, 0)),
            pl.BlockSpec((1, 1), lambda i: (0, 0)),
        ),
        out_shape=(
            jax.ShapeDtypeStruct((TOKENS, TOP_K), jnp.int32),
            jax.ShapeDtypeStruct((TOKENS, TOP_K), jnp.float32),
            jax.ShapeDtypeStruct((1, 1), jnp.float32),
        ),
    )(it, wt, aux2)


@jax.jit
def kernel(hidden_states, weight):
    hs = hidden_states.reshape(TOKENS, DIM)
    scores = _tc_scores(hs, weight)
    it, wt, aux2 = _sc_routing(scores)
    topk_idx, topk_w, aux = _tc_format(it, wt,
                                       aux2.reshape(1, NCORES * LANES))
    return (topk_idx, topk_w, aux[0, 0])


# speedup vs baseline: 1.7490x; 1.1142x over previous
import jax
import jax.numpy as jnp
from jax import lax
from jax.experimental import pallas as pl
from jax.experimental.pallas import tpu as pltpu
from jax.experimental.pallas import tpu_sc as plsc

TOP_K = 2
NUM_EXPERTS = 8
DIM = 768
ALPHA = 0.001
BSZ = 4
SEQ = 8192

TOKENS = BSZ * SEQ
LANES = 16
NCORES = 2
NSUB = 16
NW = NCORES * NSUB
TPW = TOKENS // NW
GROUPS = TPW // LANES
NACC = 2 * NUM_EXPERTS
NEG_INF = float("-inf")

BLOCK_T = 2048
GRID = TOKENS // BLOCK_T

AUX_SCALE = ALPHA * NUM_EXPERTS / (SEQ * TOP_K) / SEQ / BSZ


def _scores_body(x_ref, w_ref, s_ref):
    x = x_ref[...]
    w = w_ref[...]
    logits = lax.dot_general(
        w, x, (((1,), (1,)), ((), ())),
        preferred_element_type=jnp.float32)
    m = jnp.max(logits, axis=0, keepdims=True)
    ex = jnp.exp(logits - m)
    s_ref[...] = ex / jnp.sum(ex, axis=0, keepdims=True)


def _tc_scores(hs, weight):
    return pl.pallas_call(
        _scores_body,
        grid=(GRID,),
        in_specs=[
            pl.BlockSpec((BLOCK_T, DIM), lambda i: (i, 0)),
            pl.BlockSpec((NUM_EXPERTS, DIM), lambda i: (0, 0)),
        ],
        out_specs=pl.BlockSpec((NUM_EXPERTS, BLOCK_T), lambda i: (0, i)),
        out_shape=jax.ShapeDtypeStruct((NUM_EXPERTS, TOKENS), jnp.float32),
    )(hs, weight)


def _routing_body(scores_hbm, it_hbm, wt_hbm, aux_hbm,
                  scores_v, i1_v, i2_v, w1_v, w2_v, acc_v, red_v, out16_v,
                  shared, dma_sem):
    cid = lax.axis_index("c")
    sid = lax.axis_index("s")
    wid = cid * NSUB + sid
    base = wid * TPW

    copies = [
        pltpu.async_copy(scores_hbm.at[e, pl.ds(base, TPW)],
                         scores_v.at[pl.ds(e * TPW, TPW)], dma_sem)
        for e in range(NUM_EXPERTS)
    ]
    for c in copies:
        c.wait()

    lane = lax.iota(jnp.int32, LANES)
    zf = jnp.zeros((LANES,), jnp.float32)

    def group(g, acc):
        cnt, ssum = acc
        off = g * LANES
        p = [scores_v[pl.ds(e * TPW + off, LANES)] for e in range(NUM_EXPERTS)]

        m1 = p[0]
        i1 = jnp.zeros((LANES,), jnp.int32)
        m2 = jnp.full((LANES,), NEG_INF, jnp.float32)
        i2 = jnp.zeros((LANES,), jnp.int32)
        for e in range(1, NUM_EXPERTS):
            pe = p[e]
            ei = jnp.full((LANES,), e, jnp.int32)
            gt1 = pe > m1
            gt2 = pe > m2
            i2 = jnp.where(gt1, i1, jnp.where(gt2, ei, i2))
            m2 = jnp.where(gt1, m1, jnp.where(gt2, pe, m2))
            i1 = jnp.where(gt1, ei, i1)
            m1 = jnp.where(gt1, pe, m1)

        r = 1.0 / (m1 + m2 + 1e-20)
        sl = pl.ds(off, LANES)
        i1_v[sl] = i1
        i2_v[sl] = i2
        w1_v[sl] = m1 * r
        w2_v[sl] = m2 * r

        cnt = [cnt[e]
               + jnp.where(i1 == e, 1.0, zf)
               + jnp.where(i2 == e, 1.0, zf)
               for e in range(NUM_EXPERTS)]
        ssum = [ssum[e] + p[e] for e in range(NUM_EXPERTS)]
        return (cnt, ssum)

    init = ([zf] * NUM_EXPERTS, [zf] * NUM_EXPERTS)
    cnt, ssum = lax.fori_loop(0, GROUPS, group, init)

    pltpu.sync_copy(i1_v, it_hbm.at[0, pl.ds(base, TPW)])
    pltpu.sync_copy(i2_v, it_hbm.at[1, pl.ds(base, TPW)])
    pltpu.sync_copy(w1_v, wt_hbm.at[0, pl.ds(base, TPW)])
    pltpu.sync_copy(w2_v, wt_hbm.at[1, pl.ds(base, TPW)])

    for e in range(NUM_EXPERTS):
        acc_v[e, :] = cnt[e]
        acc_v[NUM_EXPERTS + e, :] = ssum[e]
    pltpu.sync_copy(acc_v, shared.at[sid])
    plsc.subcore_barrier()

    @pl.when(sid == 0)
    def _reduce():
        pltpu.sync_copy(shared, red_v)
        aux = jnp.float32(0.0)
        for b in range(2):
            tot = []
            for a in range(NACC):
                v = red_v[8 * b, a, :]
                for w in range(8 * b + 1, 8 * b + 8):
                    v = v + red_v[w, a, :]
                tot.append(jnp.sum(v))
            term = tot[0] * tot[NUM_EXPERTS]
            for e in range(1, NUM_EXPERTS):
                term = term + tot[e] * tot[NUM_EXPERTS + e]
            aux = aux + term
        out16_v[...] = jnp.where(lane == 0, aux * AUX_SCALE, zf)
        pltpu.sync_copy(out16_v, aux_hbm.at[pl.ds(cid * LANES, LANES)])


_sc_routing = pl.kernel(
    _routing_body,
    out_type=(
        jax.ShapeDtypeStruct((TOP_K, TOKENS), jnp.int32),
        jax.ShapeDtypeStruct((TOP_K, TOKENS), jnp.float32),
        jax.ShapeDtypeStruct((NCORES * LANES,), jnp.float32),
    ),
    mesh=plsc.VectorSubcoreMesh(core_axis_name="c", subcore_axis_name="s"),
    compiler_params=pltpu.CompilerParams(needs_layout_passes=False),
    scratch_types=[
        pltpu.VMEM((NUM_EXPERTS * TPW,), jnp.float32),
        pltpu.VMEM((TPW,), jnp.int32),
        pltpu.VMEM((TPW,), jnp.int32),
        pltpu.VMEM((TPW,), jnp.float32),
        pltpu.VMEM((TPW,), jnp.float32),
        pltpu.VMEM((NACC, LANES), jnp.float32),
        pltpu.VMEM((NSUB, NACC, LANES), jnp.float32),
        pltpu.VMEM((LANES,), jnp.float32),
        pltpu.VMEM_SHARED((NSUB, NACC, LANES), jnp.float32),
        pltpu.SemaphoreType.DMA,
    ],
)


FMT_GRID = 8
FMT_T = TOKENS // FMT_GRID
FMT_BR = FMT_T // 128


def _format_body(it_ref, wt_ref, aux_ref, io_ref, wo_ref, ao_ref):
    io_ref[...] = jnp.transpose(it_ref[...])
    wo_ref[...] = jnp.transpose(wt_ref[...])

    @pl.when(pl.program_id(0) == 0)
    def _aux():
        ao_ref[...] = jnp.sum(aux_ref[...], axis=1, keepdims=True)


def _tc_format(it, wt, aux2):
    return pl.pallas_call(
        _format_body,
        grid=(FMT_GRID,),
        in_specs=[
            pl.BlockSpec((TOP_K, FMT_T), lambda i: (0, i)),
            pl.BlockSpec((TOP_K, FMT_T), lambda i: (0, i)),
            pl.BlockSpec((1, NCORES * LANES), lambda i: (0, 0)),
        ],
        out_specs=(
            pl.BlockSpec((FMT_T, TOP_K), lambda i: (i, 0)),
            pl.BlockSpec((FMT_T, TOP_K), lambda i: (i, 0)),
            pl.BlockSpec((1, 1), lambda i: (0, 0)),
        ),
        out_shape=(
            jax.ShapeDtypeStruct((TOKENS, TOP_K), jnp.int32),
            jax.ShapeDtypeStruct((TOKENS, TOP_K), jnp.float32),
            jax.ShapeDtypeStruct((1, 1), jnp.float32),
        ),
    )(it, wt, aux2)


@jax.jit
def kernel(hidden_states, weight):
    hs = hidden_states.reshape(TOKENS, DIM)
    scores = _tc_scores(hs, weight)
    it, wt, aux2 = _sc_routing(scores)
    topk_idx, topk_w, aux = _tc_format(it, wt,
                                       aux2.reshape(1, NCORES * LANES))
    return (topk_idx, topk_w, aux[0, 0])


# speedup vs baseline: 1.7698x; 1.0119x over previous
import jax
import jax.numpy as jnp
from jax import lax
from jax.experimental import pallas as pl
from jax.experimental.pallas import tpu as pltpu
from jax.experimental.pallas import tpu_sc as plsc

TOP_K = 2
NUM_EXPERTS = 8
DIM = 768
ALPHA = 0.001
BSZ = 4
SEQ = 8192

TOKENS = BSZ * SEQ
LANES = 16
NCORES = 2
NSUB = 16
NW = NCORES * NSUB
TPW = TOKENS // NW
GROUPS = TPW // LANES
NACC = 2 * NUM_EXPERTS
NEG_INF = float("-inf")

BLOCK_T = 2048
GRID = TOKENS // BLOCK_T

AUX_SCALE = ALPHA * NUM_EXPERTS / (SEQ * TOP_K) / SEQ / BSZ


def _scores_body(x_ref, w_ref, s_ref):
    x = x_ref[...]
    w = w_ref[...]
    logits = lax.dot_general(
        w, x, (((1,), (1,)), ((), ())),
        preferred_element_type=jnp.float32)
    m = jnp.max(logits, axis=0, keepdims=True)
    ex = jnp.exp(logits - m)
    s_ref[...] = ex / jnp.sum(ex, axis=0, keepdims=True)


def _tc_scores(hs, weight):
    return pl.pallas_call(
        _scores_body,
        grid=(GRID,),
        in_specs=[
            pl.BlockSpec((BLOCK_T, DIM), lambda i: (i, 0)),
            pl.BlockSpec((NUM_EXPERTS, DIM), lambda i: (0, 0)),
        ],
        out_specs=pl.BlockSpec((NUM_EXPERTS, BLOCK_T), lambda i: (0, i)),
        out_shape=jax.ShapeDtypeStruct((NUM_EXPERTS, TOKENS), jnp.float32),
    )(hs, weight)


def _routing_body(scores_hbm, it_hbm, wt_hbm, aux_hbm,
                  scores_v, i1_v, i2_v, w1_v, w2_v, acc_v, red_v, out16_v,
                  shared, dma_sem):
    cid = lax.axis_index("c")
    sid = lax.axis_index("s")
    wid = cid * NSUB + sid
    base = wid * TPW

    copies = [
        pltpu.async_copy(scores_hbm.at[e, pl.ds(base, TPW)],
                         scores_v.at[pl.ds(e * TPW, TPW)], dma_sem)
        for e in range(NUM_EXPERTS)
    ]
    for c in copies:
        c.wait()

    lane = lax.iota(jnp.int32, LANES)
    zf = jnp.zeros((LANES,), jnp.float32)

    def group(g, acc):
        cnt, ssum = acc
        off = g * LANES
        p = [scores_v[pl.ds(e * TPW + off, LANES)] for e in range(NUM_EXPERTS)]

        m1 = p[0]
        i1 = jnp.zeros((LANES,), jnp.int32)
        m2 = jnp.full((LANES,), NEG_INF, jnp.float32)
        i2 = jnp.zeros((LANES,), jnp.int32)
        for e in range(1, NUM_EXPERTS):
            pe = p[e]
            ei = jnp.full((LANES,), e, jnp.int32)
            gt1 = pe > m1
            gt2 = pe > m2
            i2 = jnp.where(gt1, i1, jnp.where(gt2, ei, i2))
            m2 = jnp.where(gt1, m1, jnp.where(gt2, pe, m2))
            i1 = jnp.where(gt1, ei, i1)
            m1 = jnp.where(gt1, pe, m1)

        r = 1.0 / (m1 + m2 + 1e-20)
        sl = pl.ds(off, LANES)
        i1_v[sl] = i1
        i2_v[sl] = i2
        w1_v[sl] = m1 * r
        w2_v[sl] = m2 * r

        cnt = [cnt[e]
               + jnp.where(i1 == e, 1.0, zf)
               + jnp.where(i2 == e, 1.0, zf)
               for e in range(NUM_EXPERTS)]
        ssum = [ssum[e] + p[e] for e in range(NUM_EXPERTS)]
        return (cnt, ssum)

    init = ([zf] * NUM_EXPERTS, [zf] * NUM_EXPERTS)
    cnt, ssum = lax.fori_loop(0, GROUPS, group, init)

    pltpu.sync_copy(i1_v, it_hbm.at[0, pl.ds(base, TPW)])
    pltpu.sync_copy(i2_v, it_hbm.at[1, pl.ds(base, TPW)])
    pltpu.sync_copy(w1_v, wt_hbm.at[0, pl.ds(base, TPW)])
    pltpu.sync_copy(w2_v, wt_hbm.at[1, pl.ds(base, TPW)])

    for e in range(NUM_EXPERTS):
        acc_v[e, :] = cnt[e]
        acc_v[NUM_EXPERTS + e, :] = ssum[e]
    pltpu.sync_copy(acc_v, shared.at[sid])
    plsc.subcore_barrier()

    @pl.when(sid == 0)
    def _reduce():
        pltpu.sync_copy(shared, red_v)
        aux = jnp.float32(0.0)
        for b in range(2):
            tot = []
            for a in range(NACC):
                v = red_v[8 * b, a, :]
                for w in range(8 * b + 1, 8 * b + 8):
                    v = v + red_v[w, a, :]
                tot.append(jnp.sum(v))
            term = tot[0] * tot[NUM_EXPERTS]
            for e in range(1, NUM_EXPERTS):
                term = term + tot[e] * tot[NUM_EXPERTS + e]
            aux = aux + term
        out16_v[...] = jnp.where(lane == 0, aux * AUX_SCALE, zf)
        pltpu.sync_copy(out16_v, aux_hbm.at[pl.ds(cid * LANES, LANES)])


_sc_routing = pl.kernel(
    _routing_body,
    out_type=(
        jax.ShapeDtypeStruct((TOP_K, TOKENS), jnp.int32),
        jax.ShapeDtypeStruct((TOP_K, TOKENS), jnp.float32),
        jax.ShapeDtypeStruct((NCORES * LANES,), jnp.float32),
    ),
    mesh=plsc.VectorSubcoreMesh(core_axis_name="c", subcore_axis_name="s"),
    compiler_params=pltpu.CompilerParams(needs_layout_passes=False),
    scratch_types=[
        pltpu.VMEM((NUM_EXPERTS * TPW,), jnp.float32),
        pltpu.VMEM((TPW,), jnp.int32),
        pltpu.VMEM((TPW,), jnp.int32),
        pltpu.VMEM((TPW,), jnp.float32),
        pltpu.VMEM((TPW,), jnp.float32),
        pltpu.VMEM((NACC, LANES), jnp.float32),
        pltpu.VMEM((NSUB, NACC, LANES), jnp.float32),
        pltpu.VMEM((LANES,), jnp.float32),
        pltpu.VMEM_SHARED((NSUB, NACC, LANES), jnp.float32),
        pltpu.SemaphoreType.DMA,
    ],
)


FMT_GRID = 4
FMT_T = TOKENS // FMT_GRID
FMT_BR = FMT_T // 128


def _format_body(it_ref, wt_ref, aux_ref, io_ref, wo_ref, ao_ref):
    io_ref[...] = jnp.transpose(it_ref[...])
    wo_ref[...] = jnp.transpose(wt_ref[...])

    @pl.when(pl.program_id(0) == 0)
    def _aux():
        ao_ref[...] = jnp.sum(aux_ref[...], axis=1, keepdims=True)


def _tc_format(it, wt, aux2):
    return pl.pallas_call(
        _format_body,
        grid=(FMT_GRID,),
        in_specs=[
            pl.BlockSpec((TOP_K, FMT_T), lambda i: (0, i)),
            pl.BlockSpec((TOP_K, FMT_T), lambda i: (0, i)),
            pl.BlockSpec((1, NCORES * LANES), lambda i: (0, 0)),
        ],
        out_specs=(
            pl.BlockSpec((FMT_T, TOP_K), lambda i: (i, 0)),
            pl.BlockSpec((FMT_T, TOP_K), lambda i: (i, 0)),
            pl.BlockSpec((1, 1), lambda i: (0, 0)),
        ),
        out_shape=(
            jax.ShapeDtypeStruct((TOKENS, TOP_K), jnp.int32),
            jax.ShapeDtypeStruct((TOKENS, TOP_K), jnp.float32),
            jax.ShapeDtypeStruct((1, 1), jnp.float32),
        ),
    )(it, wt, aux2)


@jax.jit
def kernel(hidden_states, weight):
    hs = hidden_states.reshape(TOKENS, DIM)
    scores = _tc_scores(hs, weight)
    it, wt, aux2 = _sc_routing(scores)
    topk_idx, topk_w, aux = _tc_format(it, wt,
                                       aux2.reshape(1, NCORES * LANES))
    return (topk_idx, topk_w, aux[0, 0])
